# baseline (speedup 1.0000x reference)
import jax
import jax.numpy as jnp
from jax import lax
from jax.experimental import pallas as pl
from jax.experimental.pallas import tpu as pltpu
from jax.experimental.pallas import tpu_sc as plsc

EMB_DIM = 64
SCALE = 8.0
SEQ = 50
BATCH = 16384

NUM_CORES = 2
NUM_SUBCORES = 16
NUM_WORKERS = NUM_CORES * NUM_SUBCORES

TOTAL_TOKENS = BATCH * SEQ
PER_WORKER = TOTAL_TOKENS // NUM_WORKERS
CHUNK = 128
NUM_CHUNKS = PER_WORKER // CHUNK
BBLOCKS = BATCH // CHUNK
NBUF = 2


def _body(tokens_hbm, table_hbm, out_hbm, tok_v, idx_v, in_v, out_v, gsem, wsem):
    wid = lax.axis_index("s") * NUM_CORES + lax.axis_index("c")

    pltpu.sync_copy(tokens_hbm.at[wid], tok_v)

    lanes = lax.iota(jnp.int32, 16)

    def prep_idx(j, b):
        for g in range(CHUNK // 16):
            sl = pl.ds(g * 16, 16)
            idx_v[b, sl] = lax.shift_right_logical(tok_v[j, sl], 1)

    def gather_start(b):
        pltpu.make_async_copy(
            table_hbm.at[idx_v.at[b]], in_v.at[b], gsem.at[b]
        ).start()

    def gather_wait(b):
        pltpu.make_async_copy(
            table_hbm.at[idx_v.at[b]], in_v.at[b], gsem.at[b]
        ).wait()

    def write_start(j, b):
        c = wid * NUM_CHUNKS + j
        t = c // BBLOCKS
        b0 = (c % BBLOCKS) * CHUNK
        pltpu.make_async_copy(
            out_v.at[b], out_hbm.at[t, :, pl.ds(b0, CHUNK)], wsem.at[b]
        ).start()

    def write_wait(b):
        pltpu.make_async_copy(
            out_v.at[b], out_hbm.at[0, :, pl.ds(0, CHUNK)], wsem.at[b]
        ).wait()

    def select_scale_transpose(j, b):
        for g in range(CHUNK // 16):
            sl = pl.ds(g * 16, 16)
            tok = tok_v[j, sl]
            rows = g * 16 + lanes
            colbase = lax.bitwise_and(tok, 1) * EMB_DIM

            def feat_step(f, _):
                vals = plsc.load_gather(in_v.at[b], [rows, colbase + f])
                out_v[b, f, sl] = vals * SCALE
                return 0

            lax.fori_loop(0, EMB_DIM, feat_step, 0)

    for b in range(NBUF):
        prep_idx(b, b)
        gather_start(b)

    for b in range(NBUF):
        gather_wait(b)
        select_scale_transpose(b, b)
        write_start(b, b)
        prep_idx(NBUF + b, b)
        gather_start(b)

    def group_step(g, _):
        for b in range(NBUF):
            j = g * NBUF + b
            gather_wait(b)
            write_wait(b)
            select_scale_transpose(j, b)
            write_start(j, b)
            prep_idx(j + NBUF, b)
            gather_start(b)
        return 0

    lax.fori_loop(1, NUM_CHUNKS // NBUF - 1, group_step, 0)

    for b in range(NBUF):
        j = NUM_CHUNKS - NBUF + b
        gather_wait(b)
        write_wait(b)
        select_scale_transpose(j, b)
        write_start(j, b)

    for b in range(NBUF):
        write_wait(b)


@jax.jit
def _embed(tokens_grouped, table_pairs):
    mesh = plsc.VectorSubcoreMesh(core_axis_name="c", subcore_axis_name="s")
    out = pl.kernel(
        _body,
        out_type=jax.ShapeDtypeStruct((SEQ, EMB_DIM, BATCH), jnp.float32),
        mesh=mesh,
        scratch_types=[
            pltpu.VMEM((NUM_CHUNKS, CHUNK), jnp.int32),
            pltpu.VMEM((NBUF, CHUNK), jnp.int32),
            pltpu.VMEM((NBUF, CHUNK, CHUNK), jnp.float32),
            pltpu.VMEM((NBUF, EMB_DIM, CHUNK), jnp.float32),
            pltpu.SemaphoreType.DMA((NBUF,)),
            pltpu.SemaphoreType.DMA((NBUF,)),
        ],
        compiler_params=pltpu.CompilerParams(
            use_tc_tiling_on_sc=True, needs_layout_passes=False
        ),
    )(tokens_grouped, table_pairs)
    return out


def kernel(tokens, table):
    tokens_lin = tokens.astype(jnp.int32).T.reshape(-1)
    grouped = tokens_lin.reshape(NUM_WORKERS, NUM_CHUNKS, CHUNK)
    table_pairs = table.reshape(500000, 128)
    out = _embed(grouped, table_pairs)
    return jnp.transpose(out, (2, 0, 1))
